# baseline (speedup 1.0000x reference)
import jax
import jax.numpy as jnp
from jax import lax
from jax.experimental import pallas as pl
from jax.experimental.pallas import tpu as pltpu
from jax.experimental.pallas import tpu_sc as plsc

N_NODES = 50000
D = 256
B = 1024
S1 = 25
S2 = 10
MAX_DEG = 32
NCLS = 64

NW = 32
BT = B // NW
L1T = BT * S1
CH = 16
NCH = L1T // CH
L2CH = CH * S2
LANES = 16
ADJ_FOLD = 128 // MAX_DEG


def _sc_body(bn_hbm, adj_hbm, feat_hbm, featb_hbm, h0_hbm, h1_hbm, m2_hbm,
             ids_b, rowb, adj_b, h0_buf, l1ids, m2_buf0, m2_buf1,
             rowc0, rowc1, colc0, colc1, adj_c0, adj_c1,
             h1_buf0, h1_buf1, l2c0, l2c1, f2_buf0, f2_buf1,
             sem_a0, sem_a1, sem_f0, sem_f1, sem_h0, sem_h1,
             sem_ho0, sem_ho1, sem_mo0, sem_mo1):
    rowc = (rowc0, rowc1)
    colc = (colc0, colc1)
    adj_c = (adj_c0, adj_c1)
    h1_buf = (h1_buf0, h1_buf1)
    m2_buf = (m2_buf0, m2_buf1)
    l2c = (l2c0, l2c1)
    f2_buf = (f2_buf0, f2_buf1)
    sem_a = (sem_a0, sem_a1)
    sem_f = (sem_f0, sem_f1)
    sem_h = (sem_h0, sem_h1)
    sem_ho = (sem_ho0, sem_ho1)
    sem_mo = (sem_mo0, sem_mo1)

    wid = lax.axis_index("s") * 2 + lax.axis_index("c")
    base_b = pl.multiple_of(wid * BT, BT)
    iot = lax.iota(jnp.int32, LANES)

    pltpu.sync_copy(bn_hbm.at[pl.ds(base_b, BT)], ids_b)
    for j in range(BT // LANES):
        v = ids_b[pl.ds(j * LANES, LANES)]
        rowb[pl.ds(j * LANES, LANES)] = v // ADJ_FOLD
    cp_adj_b = pltpu.async_copy(adj_hbm.at[rowb], adj_b, sem_a0)
    cp_h0 = pltpu.async_copy(feat_hbm.at[ids_b], h0_buf, sem_h0)
    cp_adj_b.wait()
    for j in range(L1T // LANES):
        p = j * LANES + iot
        s = p // S1
        node = plsc.load_gather(ids_b, [s])
        col = (node % ADJ_FOLD) * MAX_DEG + p % S1
        l1ids[pl.ds(j * LANES, LANES)] = plsc.load_gather(adj_b, [s, col])
    cp_h0.wait()
    pltpu.sync_copy(h0_buf, h0_hbm.at[pl.ds(base_b, BT)])

    def start_adj(c, b):
        cs = jnp.minimum(c, NCH - 1)
        v = l1ids[pl.ds(cs * CH, CH)]
        rowc[b][...] = v // ADJ_FOLD
        colc[b][...] = (v % ADJ_FOLD) * MAX_DEG
        pltpu.async_copy(adj_hbm.at[rowc[b]], adj_c[b], sem_a[b])

    def wait_adj(b):
        pltpu.make_async_copy(adj_hbm.at[rowc[b]], adj_c[b], sem_a[b]).wait()

    def start_f2(c, b, wait_ho=True):
        cs = jnp.minimum(c, NCH - 1)
        for j in range(L2CH // LANES):
            p = j * LANES + iot
            s = p // S2
            col = plsc.load_gather(colc[b], [s]) + p % S2
            l2c[b][pl.ds(j * LANES, LANES)] = plsc.load_gather(adj_c[b], [s, col])
        pltpu.async_copy(featb_hbm.at[l2c[b]], f2_buf[b], sem_f[b])
        if wait_ho:
            pltpu.make_async_copy(h1_hbm.at[pl.ds(0, CH)], h1_buf[b],
                                  sem_ho[b]).wait()
        pltpu.async_copy(feat_hbm.at[l1ids.at[pl.ds(cs * CH, CH)]],
                         h1_buf[b], sem_h[b])

    def finish(c, b, wait_mo=True):
        row0 = pl.multiple_of(wid * L1T + c * CH, CH)
        pltpu.make_async_copy(feat_hbm.at[l1ids.at[pl.ds(c * CH, CH)]],
                              h1_buf[b], sem_h[b]).wait()
        pltpu.async_copy(h1_buf[b], h1_hbm.at[pl.ds(row0, CH)], sem_ho[b])
        if wait_mo:
            pltpu.make_async_copy(m2_hbm.at[pl.ds(0, CH)], m2_buf[b],
                                  sem_mo[b]).wait()
        pltpu.make_async_copy(featb_hbm.at[l2c[b]], f2_buf[b], sem_f[b]).wait()

        hi_mask = jnp.int32(-65536)
        inv = jnp.float32(1.0 / S2)

        def seg(s2, carry2):
            s2v = s2 + iot * 0
            for j in range(D // 32):
                r0 = s2 * S2
                vi = f2_buf[b][r0, pl.ds(j * LANES, LANES)]
                acc_lo = plsc.bitcast(vi << 16, jnp.float32)
                acc_hi = plsc.bitcast(vi & hi_mask, jnp.float32)
                for t in range(1, S2):
                    vi = f2_buf[b][r0 + t, pl.ds(j * LANES, LANES)]
                    acc_lo = acc_lo + plsc.bitcast(vi << 16, jnp.float32)
                    acc_hi = acc_hi + plsc.bitcast(vi & hi_mask, jnp.float32)
                cb = j * 32
                plsc.store_scatter(m2_buf[b], [s2v, cb + 2 * iot],
                                   acc_lo * inv)
                plsc.store_scatter(m2_buf[b], [s2v, cb + 1 + 2 * iot],
                                   acc_hi * inv)
            return carry2

        lax.fori_loop(0, CH, seg, 0)
        pltpu.async_copy(m2_buf[b], m2_hbm.at[pl.ds(row0, CH)], sem_mo[b])

    start_adj(0, 0)
    wait_adj(0)
    start_f2(0, 0, wait_ho=False)
    start_adj(1, 1)

    def pair_body(c0, wait_first):
        wait_adj(1)
        start_f2(c0 + 1, 1, wait_ho=wait_first)
        start_adj(c0 + 2, 0)
        finish(c0, 0, wait_mo=wait_first)
        wait_adj(0)
        start_f2(c0 + 2, 0)
        start_adj(c0 + 3, 1)
        finish(c0 + 1, 1, wait_mo=wait_first)

    pair_body(jnp.int32(0), False)

    def pair(g, carry):
        pair_body(g * 2, True)
        return carry

    lax.fori_loop(1, NCH // 2, pair, 0)
    pltpu.make_async_copy(feat_hbm.at[l1ids.at[pl.ds((NCH - 1) * CH, CH)]],
                          h1_buf[0], sem_h[0]).wait()
    pltpu.make_async_copy(featb_hbm.at[l2c[0]], f2_buf[0], sem_f[0]).wait()
    wait_adj(1)
    pltpu.make_async_copy(h1_hbm.at[pl.ds(0, CH)], h1_buf[1], sem_ho[1]).wait()
    pltpu.make_async_copy(m2_hbm.at[pl.ds(0, CH)], m2_buf[0], sem_mo[0]).wait()
    pltpu.make_async_copy(m2_hbm.at[pl.ds(0, CH)], m2_buf[1], sem_mo[1]).wait()


def _sc_gather(batch_nodes, adj_packed, features, features_bf):
    mesh = plsc.VectorSubcoreMesh(core_axis_name="c", subcore_axis_name="s")
    f32, i32 = jnp.float32, jnp.int32
    fn = pl.kernel(
        _sc_body,
        mesh=mesh,
        compiler_params=pltpu.CompilerParams(needs_layout_passes=False),
        out_type=[
            jax.ShapeDtypeStruct((B, D), f32),
            jax.ShapeDtypeStruct((B * S1, D), f32),
            jax.ShapeDtypeStruct((B * S1, D), f32),
        ],
        scratch_types=[
            pltpu.VMEM((BT,), i32),
            pltpu.VMEM((BT,), i32),
            pltpu.VMEM((BT, 128), i32),
            pltpu.VMEM((BT, D), f32),
            pltpu.VMEM((L1T,), i32),
            pltpu.VMEM((CH, D), f32),
            pltpu.VMEM((CH, D), f32),
            pltpu.VMEM((CH,), i32),
            pltpu.VMEM((CH,), i32),
            pltpu.VMEM((CH,), i32),
            pltpu.VMEM((CH,), i32),
            pltpu.VMEM((CH, 128), i32),
            pltpu.VMEM((CH, 128), i32),
            pltpu.VMEM((CH, D), f32),
            pltpu.VMEM((CH, D), f32),
            pltpu.VMEM((L2CH,), i32),
            pltpu.VMEM((L2CH,), i32),
            pltpu.VMEM((L2CH, D // 2), i32),
            pltpu.VMEM((L2CH, D // 2), i32),
            pltpu.SemaphoreType.DMA,
            pltpu.SemaphoreType.DMA,
            pltpu.SemaphoreType.DMA,
            pltpu.SemaphoreType.DMA,
            pltpu.SemaphoreType.DMA,
            pltpu.SemaphoreType.DMA,
            pltpu.SemaphoreType.DMA,
            pltpu.SemaphoreType.DMA,
            pltpu.SemaphoreType.DMA,
            pltpu.SemaphoreType.DMA,
        ],
    )
    return fn(batch_nodes, adj_packed, features, features_bf)


def _t1_body(h1_ref, m2_ref, ws_ref, wn_ref, b_ref, m1_ref, p_ref):
    h1 = h1_ref[...]
    a1 = h1 @ ws_ref[...] + m2_ref[...] @ wn_ref[...] + b_ref[...]
    a1 = jnp.maximum(a1, 0.0)
    rows = lax.broadcasted_iota(jnp.int32, (BT, L1T), 0)
    segs = lax.broadcasted_iota(jnp.int32, (BT, L1T), 1) // S1
    pool = jnp.where(rows == segs, jnp.float32(1.0 / S1), jnp.float32(0.0))
    p_ref[...] = jnp.dot(pool, a1, preferred_element_type=jnp.float32)
    m1_ref[...] = jnp.dot(pool, h1, preferred_element_type=jnp.float32)


def _t2_body(h0_ref, m1_ref, p_ref, ws1_ref, wn1_ref, b1_ref,
             ws2_ref, wn2_ref, b2_ref, fcw_ref, fcb_ref, out_ref):
    a0 = h0_ref[...] @ ws1_ref[...] + m1_ref[...] @ wn1_ref[...] + b1_ref[...]
    a0 = jnp.maximum(a0, 0.0)
    e = a0 @ ws2_ref[...] + p_ref[...] @ wn2_ref[...] + b2_ref[...]
    e = jnp.maximum(e, 0.0)
    logits = jnp.maximum(e @ fcw_ref[...] + fcb_ref[...], 0.0)
    m = jnp.max(logits, axis=1, keepdims=True)
    ex = jnp.exp(logits - m)
    out_ref[...] = ex / jnp.sum(ex, axis=1, keepdims=True)


def kernel(batchNodes, features, adj, W_self1, W_neigh1, b1,
           W_self2, W_neigh2, b2, fcW, fcb):
    f32 = jnp.float32
    adj_packed = adj.reshape(N_NODES * MAX_DEG // 128, 128)
    features_bf = jax.lax.bitcast_convert_type(
        features.astype(jnp.bfloat16).reshape(N_NODES, D // 2, 2),
        jnp.int32)
    h0, h1, m2 = _sc_gather(batchNodes, adj_packed, features, features_bf)

    b1_2d = b1.reshape(1, D)
    b2_2d = b2.reshape(1, D)
    fcb_2d = fcb.reshape(1, NCLS)

    m1, p = pl.pallas_call(
        _t1_body,
        grid=(NW,),
        in_specs=[
            pl.BlockSpec((L1T, D), lambda i: (i, 0)),
            pl.BlockSpec((L1T, D), lambda i: (i, 0)),
            pl.BlockSpec((D, D), lambda i: (0, 0)),
            pl.BlockSpec((D, D), lambda i: (0, 0)),
            pl.BlockSpec((1, D), lambda i: (0, 0)),
        ],
        out_specs=[
            pl.BlockSpec((BT, D), lambda i: (i, 0)),
            pl.BlockSpec((BT, D), lambda i: (i, 0)),
        ],
        out_shape=[
            jax.ShapeDtypeStruct((B, D), f32),
            jax.ShapeDtypeStruct((B, D), f32),
        ],
    )(h1, m2, W_self1, W_neigh1, b1_2d)

    out = pl.pallas_call(
        _t2_body,
        out_shape=jax.ShapeDtypeStruct((B, NCLS), f32),
    )(h0, m1, p, W_self1, W_neigh1, b1_2d, W_self2, W_neigh2, b2_2d,
      fcW, fcb_2d)
    return out


# speedup vs baseline: 2.5888x; 2.5888x over previous
import jax
import jax.numpy as jnp
from jax import lax
from jax.experimental import pallas as pl
from jax.experimental.pallas import tpu as pltpu
from jax.experimental.pallas import tpu_sc as plsc

N_NODES = 50000
D = 256
B = 1024
S1 = 25
S2 = 10
MAX_DEG = 32
NCLS = 64

NW = 32
BT = B // NW
L1T = BT * S1
CH = 16
NCH = L1T // CH
L2CH = CH * S2
LANES = 16
ADJ_FOLD = 128 // MAX_DEG


def _sc_body(bn_hbm, adj_hbm, feat_hbm, featb_hbm, h0_hbm, h1_hbm, m2_hbm,
             ids_b, rowb, adj_b, h0_buf, l1ids, m2_buf0, m2_buf1,
             rowc0, rowc1, colc0, colc1, adj_c0, adj_c1,
             h1_buf0, h1_buf1, l2c0, l2c1, f2_buf0, f2_buf1,
             sem_a0, sem_a1, sem_f0, sem_f1, sem_h0, sem_h1,
             sem_ho0, sem_ho1, sem_mo0, sem_mo1):
    rowc = (rowc0, rowc1)
    colc = (colc0, colc1)
    adj_c = (adj_c0, adj_c1)
    h1_buf = (h1_buf0, h1_buf1)
    m2_buf = (m2_buf0, m2_buf1)
    l2c = (l2c0, l2c1)
    f2_buf = (f2_buf0, f2_buf1)
    sem_a = (sem_a0, sem_a1)
    sem_f = (sem_f0, sem_f1)
    sem_h = (sem_h0, sem_h1)
    sem_ho = (sem_ho0, sem_ho1)
    sem_mo = (sem_mo0, sem_mo1)

    wid = lax.axis_index("s") * 2 + lax.axis_index("c")
    base_b = pl.multiple_of(wid * BT, BT)
    iot = lax.iota(jnp.int32, LANES)

    pltpu.sync_copy(bn_hbm.at[pl.ds(base_b, BT)], ids_b)
    for j in range(BT // LANES):
        v = ids_b[pl.ds(j * LANES, LANES)]
        rowb[pl.ds(j * LANES, LANES)] = v // ADJ_FOLD
    cp_adj_b = pltpu.async_copy(adj_hbm.at[rowb], adj_b, sem_a0)
    cp_h0 = pltpu.async_copy(feat_hbm.at[ids_b], h0_buf, sem_h0)
    cp_adj_b.wait()
    for j in range(L1T // LANES):
        p = j * LANES + iot
        s = p // S1
        node = plsc.load_gather(ids_b, [s])
        col = (node % ADJ_FOLD) * MAX_DEG + p % S1
        l1ids[pl.ds(j * LANES, LANES)] = plsc.load_gather(adj_b, [s, col])
    cp_h0.wait()
    pltpu.sync_copy(h0_buf, h0_hbm.at[pl.ds(base_b, BT)])

    def start_adj(c, b):
        cs = jnp.minimum(c, NCH - 1)
        v = l1ids[pl.ds(cs * CH, CH)]
        rowc[b][...] = v // ADJ_FOLD
        colc[b][...] = (v % ADJ_FOLD) * MAX_DEG
        pltpu.async_copy(adj_hbm.at[rowc[b]], adj_c[b], sem_a[b])

    def wait_adj(b):
        pltpu.make_async_copy(adj_hbm.at[rowc[b]], adj_c[b], sem_a[b]).wait()

    def start_f2(c, b, wait_ho=True):
        cs = jnp.minimum(c, NCH - 1)
        for j in range(L2CH // LANES):
            p = j * LANES + iot
            s = p // S2
            col = plsc.load_gather(colc[b], [s]) + p % S2
            l2c[b][pl.ds(j * LANES, LANES)] = plsc.load_gather(adj_c[b], [s, col])
        pltpu.async_copy(featb_hbm.at[l2c[b]], f2_buf[b], sem_f[b])
        if wait_ho:
            pltpu.make_async_copy(h1_hbm.at[pl.ds(0, CH)], h1_buf[b],
                                  sem_ho[b]).wait()
        pltpu.async_copy(feat_hbm.at[l1ids.at[pl.ds(cs * CH, CH)]],
                         h1_buf[b], sem_h[b])

    def finish(c, b, wait_mo=True):
        row0 = pl.multiple_of(wid * L1T + c * CH, CH)
        pltpu.make_async_copy(feat_hbm.at[l1ids.at[pl.ds(c * CH, CH)]],
                              h1_buf[b], sem_h[b]).wait()
        pltpu.async_copy(h1_buf[b], h1_hbm.at[pl.ds(row0, CH)], sem_ho[b])
        if wait_mo:
            pltpu.make_async_copy(m2_hbm.at[pl.ds(0, CH)], m2_buf[b],
                                  sem_mo[b]).wait()
        pltpu.make_async_copy(featb_hbm.at[l2c[b]], f2_buf[b], sem_f[b]).wait()

        hi_mask = jnp.int32(-65536)
        inv = jnp.float32(1.0 / S2)

        def seg(s2, carry2):
            for j in range(D // 32):
                r0 = s2 * S2
                vi = f2_buf[b][r0, pl.ds(j * LANES, LANES)]
                acc_lo = plsc.bitcast(vi << 16, jnp.float32)
                acc_hi = plsc.bitcast(vi & hi_mask, jnp.float32)
                for t in range(1, S2):
                    vi = f2_buf[b][r0 + t, pl.ds(j * LANES, LANES)]
                    acc_lo = acc_lo + plsc.bitcast(vi << 16, jnp.float32)
                    acc_hi = acc_hi + plsc.bitcast(vi & hi_mask, jnp.float32)
                m2_buf[b][s2, pl.ds(j * LANES, LANES)] = acc_lo * inv
                m2_buf[b][s2, pl.ds(D // 2 + j * LANES, LANES)] = acc_hi * inv
            return carry2

        lax.fori_loop(0, CH, seg, 0)
        pltpu.async_copy(m2_buf[b], m2_hbm.at[pl.ds(row0, CH)], sem_mo[b])

    start_adj(0, 0)
    wait_adj(0)
    start_f2(0, 0, wait_ho=False)
    start_adj(1, 1)

    def pair_body(c0, wait_first):
        wait_adj(1)
        start_f2(c0 + 1, 1, wait_ho=wait_first)
        start_adj(c0 + 2, 0)
        finish(c0, 0, wait_mo=wait_first)
        wait_adj(0)
        start_f2(c0 + 2, 0)
        start_adj(c0 + 3, 1)
        finish(c0 + 1, 1, wait_mo=wait_first)

    pair_body(jnp.int32(0), False)

    def pair(g, carry):
        pair_body(g * 2, True)
        return carry

    lax.fori_loop(1, NCH // 2, pair, 0)
    pltpu.make_async_copy(feat_hbm.at[l1ids.at[pl.ds((NCH - 1) * CH, CH)]],
                          h1_buf[0], sem_h[0]).wait()
    pltpu.make_async_copy(featb_hbm.at[l2c[0]], f2_buf[0], sem_f[0]).wait()
    wait_adj(1)
    pltpu.make_async_copy(h1_hbm.at[pl.ds(0, CH)], h1_buf[1], sem_ho[1]).wait()
    pltpu.make_async_copy(m2_hbm.at[pl.ds(0, CH)], m2_buf[0], sem_mo[0]).wait()
    pltpu.make_async_copy(m2_hbm.at[pl.ds(0, CH)], m2_buf[1], sem_mo[1]).wait()


def _sc_gather(batch_nodes, adj_packed, features, features_bf):
    mesh = plsc.VectorSubcoreMesh(core_axis_name="c", subcore_axis_name="s")
    f32, i32 = jnp.float32, jnp.int32
    fn = pl.kernel(
        _sc_body,
        mesh=mesh,
        compiler_params=pltpu.CompilerParams(needs_layout_passes=False),
        out_type=[
            jax.ShapeDtypeStruct((B, D), f32),
            jax.ShapeDtypeStruct((B * S1, D), f32),
            jax.ShapeDtypeStruct((B * S1, D), f32),
        ],
        scratch_types=[
            pltpu.VMEM((BT,), i32),
            pltpu.VMEM((BT,), i32),
            pltpu.VMEM((BT, 128), i32),
            pltpu.VMEM((BT, D), f32),
            pltpu.VMEM((L1T,), i32),
            pltpu.VMEM((CH, D), f32),
            pltpu.VMEM((CH, D), f32),
            pltpu.VMEM((CH,), i32),
            pltpu.VMEM((CH,), i32),
            pltpu.VMEM((CH,), i32),
            pltpu.VMEM((CH,), i32),
            pltpu.VMEM((CH, 128), i32),
            pltpu.VMEM((CH, 128), i32),
            pltpu.VMEM((CH, D), f32),
            pltpu.VMEM((CH, D), f32),
            pltpu.VMEM((L2CH,), i32),
            pltpu.VMEM((L2CH,), i32),
            pltpu.VMEM((L2CH, D // 2), i32),
            pltpu.VMEM((L2CH, D // 2), i32),
            pltpu.SemaphoreType.DMA,
            pltpu.SemaphoreType.DMA,
            pltpu.SemaphoreType.DMA,
            pltpu.SemaphoreType.DMA,
            pltpu.SemaphoreType.DMA,
            pltpu.SemaphoreType.DMA,
            pltpu.SemaphoreType.DMA,
            pltpu.SemaphoreType.DMA,
            pltpu.SemaphoreType.DMA,
            pltpu.SemaphoreType.DMA,
        ],
    )
    return fn(batch_nodes, adj_packed, features, features_bf)


def _t1_body(h1_ref, m2_ref, ws_ref, wn_ref, b_ref, m1_ref, p_ref):
    h1 = h1_ref[...]
    a1 = h1 @ ws_ref[...] + m2_ref[...] @ wn_ref[...] + b_ref[...]
    a1 = jnp.maximum(a1, 0.0)
    rows = lax.broadcasted_iota(jnp.int32, (BT, L1T), 0)
    segs = lax.broadcasted_iota(jnp.int32, (BT, L1T), 1) // S1
    pool = jnp.where(rows == segs, jnp.float32(1.0 / S1), jnp.float32(0.0))
    p_ref[...] = jnp.dot(pool, a1, preferred_element_type=jnp.float32)
    m1_ref[...] = jnp.dot(pool, h1, preferred_element_type=jnp.float32)


def _t2_body(h0_ref, m1_ref, p_ref, ws1_ref, wn1_ref, b1_ref,
             ws2_ref, wn2_ref, b2_ref, fcw_ref, fcb_ref, out_ref):
    a0 = h0_ref[...] @ ws1_ref[...] + m1_ref[...] @ wn1_ref[...] + b1_ref[...]
    a0 = jnp.maximum(a0, 0.0)
    e = a0 @ ws2_ref[...] + p_ref[...] @ wn2_ref[...] + b2_ref[...]
    e = jnp.maximum(e, 0.0)
    logits = jnp.maximum(e @ fcw_ref[...] + fcb_ref[...], 0.0)
    m = jnp.max(logits, axis=1, keepdims=True)
    ex = jnp.exp(logits - m)
    out_ref[...] = ex / jnp.sum(ex, axis=1, keepdims=True)


def kernel(batchNodes, features, adj, W_self1, W_neigh1, b1,
           W_self2, W_neigh2, b2, fcW, fcb):
    f32 = jnp.float32
    adj_packed = adj.reshape(N_NODES * MAX_DEG // 128, 128)
    fu = jax.lax.bitcast_convert_type(features, jnp.uint32)
    rnd = jnp.uint32(0x8000)
    features_bf = jax.lax.bitcast_convert_type(
        ((fu[:, : D // 2] + rnd) >> 16)
        | ((fu[:, D // 2:] + rnd) & jnp.uint32(0xFFFF0000)),
        jnp.int32)
    h0, h1, m2 = _sc_gather(batchNodes, adj_packed, features, features_bf)

    b1_2d = b1.reshape(1, D)
    b2_2d = b2.reshape(1, D)
    fcb_2d = fcb.reshape(1, NCLS)

    m1, p = pl.pallas_call(
        _t1_body,
        grid=(NW,),
        in_specs=[
            pl.BlockSpec((L1T, D), lambda i: (i, 0)),
            pl.BlockSpec((L1T, D), lambda i: (i, 0)),
            pl.BlockSpec((D, D), lambda i: (0, 0)),
            pl.BlockSpec((D, D), lambda i: (0, 0)),
            pl.BlockSpec((1, D), lambda i: (0, 0)),
        ],
        out_specs=[
            pl.BlockSpec((BT, D), lambda i: (i, 0)),
            pl.BlockSpec((BT, D), lambda i: (i, 0)),
        ],
        out_shape=[
            jax.ShapeDtypeStruct((B, D), f32),
            jax.ShapeDtypeStruct((B, D), f32),
        ],
    )(h1, m2, W_self1, W_neigh1, b1_2d)

    out = pl.pallas_call(
        _t2_body,
        out_shape=jax.ShapeDtypeStruct((B, NCLS), f32),
    )(h0, m1, p, W_self1, W_neigh1, b1_2d, W_self2, W_neigh2, b2_2d,
      fcW, fcb_2d)
    return out


# speedup vs baseline: 2.6220x; 1.0128x over previous
import jax
import jax.numpy as jnp
from jax import lax
from jax.experimental import pallas as pl
from jax.experimental.pallas import tpu as pltpu
from jax.experimental.pallas import tpu_sc as plsc

N_NODES = 50000
D = 256
B = 1024
S1 = 25
S2 = 10
MAX_DEG = 32
NCLS = 64

NW = 32
BT = B // NW
L1T = BT * S1
CH = 16
NCH = L1T // CH
L2CH = CH * S2
LANES = 16
ADJ_FOLD = 128 // MAX_DEG


def _sc_body(bn_hbm, adj_hbm, feat_hbm, featb_hbm, h0_hbm, h1_hbm, m2_hbm,
             ids_b, rowb, adj_b, h0_buf, l1ids, m2_buf0, m2_buf1,
             rowc0, rowc1, colc0, colc1, adj_c0, adj_c1,
             h1_buf0, h1_buf1, l2c0, l2c1, f2_buf0, f2_buf1,
             sem_a0, sem_a1, sem_f0, sem_f1, sem_h0, sem_h1,
             sem_ho0, sem_ho1, sem_mo0, sem_mo1):
    rowc = (rowc0, rowc1)
    colc = (colc0, colc1)
    adj_c = (adj_c0, adj_c1)
    h1_buf = (h1_buf0, h1_buf1)
    m2_buf = (m2_buf0, m2_buf1)
    l2c = (l2c0, l2c1)
    f2_buf = (f2_buf0, f2_buf1)
    sem_a = (sem_a0, sem_a1)
    sem_f = (sem_f0, sem_f1)
    sem_h = (sem_h0, sem_h1)
    sem_ho = (sem_ho0, sem_ho1)
    sem_mo = (sem_mo0, sem_mo1)

    wid = lax.axis_index("s") * 2 + lax.axis_index("c")
    base_b = pl.multiple_of(wid * BT, BT)
    iot = lax.iota(jnp.int32, LANES)

    pltpu.sync_copy(bn_hbm.at[pl.ds(base_b, BT)], ids_b)
    for j in range(BT // LANES):
        v = ids_b[pl.ds(j * LANES, LANES)]
        rowb[pl.ds(j * LANES, LANES)] = v // ADJ_FOLD
    cp_adj_b = pltpu.async_copy(adj_hbm.at[rowb], adj_b, sem_a0)
    cp_h0 = pltpu.async_copy(feat_hbm.at[ids_b], h0_buf, sem_h0)
    cp_adj_b.wait()
    for j in range(L1T // LANES):
        p = j * LANES + iot
        s = p // S1
        node = plsc.load_gather(ids_b, [s])
        col = (node % ADJ_FOLD) * MAX_DEG + p % S1
        l1ids[pl.ds(j * LANES, LANES)] = plsc.load_gather(adj_b, [s, col])
    cp_h0.wait()
    pltpu.sync_copy(h0_buf, h0_hbm.at[pl.ds(base_b, BT)])

    def start_adj(c, b):
        cs = jnp.minimum(c, NCH - 1)
        v = l1ids[pl.ds(cs * CH, CH)]
        rowc[b][...] = v // ADJ_FOLD
        colc[b][...] = (v % ADJ_FOLD) * MAX_DEG
        pltpu.async_copy(adj_hbm.at[rowc[b]], adj_c[b], sem_a[b])

    def wait_adj(b):
        pltpu.make_async_copy(adj_hbm.at[rowc[b]], adj_c[b], sem_a[b]).wait()

    def start_f2(c, b, wait_ho=True):
        cs = jnp.minimum(c, NCH - 1)
        for j in range(L2CH // LANES):
            p = j * LANES + iot
            s = p // S2
            col = plsc.load_gather(colc[b], [s]) + p % S2
            l2c[b][pl.ds(j * LANES, LANES)] = plsc.load_gather(adj_c[b], [s, col])
        pltpu.async_copy(featb_hbm.at[l2c[b]], f2_buf[b], sem_f[b])
        if wait_ho:
            pltpu.make_async_copy(h1_hbm.at[pl.ds(0, CH)], h1_buf[b],
                                  sem_ho[b]).wait()
        pltpu.async_copy(featb_hbm.at[l1ids.at[pl.ds(cs * CH, CH)]],
                         h1_buf[b], sem_h[b])

    def finish(c, b, wait_mo=True):
        row0 = pl.multiple_of(wid * L1T + c * CH, CH)
        pltpu.make_async_copy(featb_hbm.at[l1ids.at[pl.ds(c * CH, CH)]],
                              h1_buf[b], sem_h[b]).wait()
        pltpu.async_copy(h1_buf[b], h1_hbm.at[pl.ds(row0, CH)], sem_ho[b])
        if wait_mo:
            pltpu.make_async_copy(m2_hbm.at[pl.ds(0, CH)], m2_buf[b],
                                  sem_mo[b]).wait()
        pltpu.make_async_copy(featb_hbm.at[l2c[b]], f2_buf[b], sem_f[b]).wait()

        hi_mask = jnp.int32(-65536)
        inv = jnp.float32(1.0 / S2)

        def seg(s2, carry2):
            for j in range(D // 32):
                r0 = s2 * S2
                vi = f2_buf[b][r0, pl.ds(j * LANES, LANES)]
                acc_lo = plsc.bitcast(vi << 16, jnp.float32)
                acc_hi = plsc.bitcast(vi & hi_mask, jnp.float32)
                for t in range(1, S2):
                    vi = f2_buf[b][r0 + t, pl.ds(j * LANES, LANES)]
                    acc_lo = acc_lo + plsc.bitcast(vi << 16, jnp.float32)
                    acc_hi = acc_hi + plsc.bitcast(vi & hi_mask, jnp.float32)
                m2_buf[b][s2, pl.ds(j * LANES, LANES)] = acc_lo * inv
                m2_buf[b][s2, pl.ds(D // 2 + j * LANES, LANES)] = acc_hi * inv
            return carry2

        lax.fori_loop(0, CH, seg, 0)
        pltpu.async_copy(m2_buf[b], m2_hbm.at[pl.ds(row0, CH)], sem_mo[b])

    start_adj(0, 0)
    wait_adj(0)
    start_f2(0, 0, wait_ho=False)
    start_adj(1, 1)

    def pair_body(c0, wait_first):
        wait_adj(1)
        start_f2(c0 + 1, 1, wait_ho=wait_first)
        start_adj(c0 + 2, 0)
        finish(c0, 0, wait_mo=wait_first)
        wait_adj(0)
        start_f2(c0 + 2, 0)
        start_adj(c0 + 3, 1)
        finish(c0 + 1, 1, wait_mo=wait_first)

    pair_body(jnp.int32(0), False)

    def pair(g, carry):
        pair_body(g * 2, True)
        return carry

    lax.fori_loop(1, NCH // 2, pair, 0)
    pltpu.make_async_copy(featb_hbm.at[l1ids.at[pl.ds((NCH - 1) * CH, CH)]],
                          h1_buf[0], sem_h[0]).wait()
    pltpu.make_async_copy(featb_hbm.at[l2c[0]], f2_buf[0], sem_f[0]).wait()
    wait_adj(1)
    pltpu.make_async_copy(h1_hbm.at[pl.ds(0, CH)], h1_buf[1], sem_ho[1]).wait()
    pltpu.make_async_copy(m2_hbm.at[pl.ds(0, CH)], m2_buf[0], sem_mo[0]).wait()
    pltpu.make_async_copy(m2_hbm.at[pl.ds(0, CH)], m2_buf[1], sem_mo[1]).wait()


def _sc_gather(batch_nodes, adj_packed, features, features_bf):
    mesh = plsc.VectorSubcoreMesh(core_axis_name="c", subcore_axis_name="s")
    f32, i32 = jnp.float32, jnp.int32
    fn = pl.kernel(
        _sc_body,
        mesh=mesh,
        compiler_params=pltpu.CompilerParams(needs_layout_passes=False),
        out_type=[
            jax.ShapeDtypeStruct((B, D), f32),
            jax.ShapeDtypeStruct((B * S1, D // 2), i32),
            jax.ShapeDtypeStruct((B * S1, D), f32),
        ],
        scratch_types=[
            pltpu.VMEM((BT,), i32),
            pltpu.VMEM((BT,), i32),
            pltpu.VMEM((BT, 128), i32),
            pltpu.VMEM((BT, D), f32),
            pltpu.VMEM((L1T,), i32),
            pltpu.VMEM((CH, D), f32),
            pltpu.VMEM((CH, D), f32),
            pltpu.VMEM((CH,), i32),
            pltpu.VMEM((CH,), i32),
            pltpu.VMEM((CH,), i32),
            pltpu.VMEM((CH,), i32),
            pltpu.VMEM((CH, 128), i32),
            pltpu.VMEM((CH, 128), i32),
            pltpu.VMEM((CH, D // 2), i32),
            pltpu.VMEM((CH, D // 2), i32),
            pltpu.VMEM((L2CH,), i32),
            pltpu.VMEM((L2CH,), i32),
            pltpu.VMEM((L2CH, D // 2), i32),
            pltpu.VMEM((L2CH, D // 2), i32),
            pltpu.SemaphoreType.DMA,
            pltpu.SemaphoreType.DMA,
            pltpu.SemaphoreType.DMA,
            pltpu.SemaphoreType.DMA,
            pltpu.SemaphoreType.DMA,
            pltpu.SemaphoreType.DMA,
            pltpu.SemaphoreType.DMA,
            pltpu.SemaphoreType.DMA,
            pltpu.SemaphoreType.DMA,
            pltpu.SemaphoreType.DMA,
        ],
    )
    return fn(batch_nodes, adj_packed, features, features_bf)


def _t1_body(h1p_ref, m2_ref, ws_ref, wn_ref, b_ref, m1_ref, p_ref):
    vi = h1p_ref[...]
    h1 = jnp.concatenate(
        [
            jax.lax.bitcast_convert_type(vi << 16, jnp.float32),
            jax.lax.bitcast_convert_type(vi & jnp.int32(-65536), jnp.float32),
        ],
        axis=1,
    )
    a1 = h1 @ ws_ref[...] + m2_ref[...] @ wn_ref[...] + b_ref[...]
    a1 = jnp.maximum(a1, 0.0)
    rows = lax.broadcasted_iota(jnp.int32, (BT, L1T), 0)
    segs = lax.broadcasted_iota(jnp.int32, (BT, L1T), 1) // S1
    pool = jnp.where(rows == segs, jnp.float32(1.0 / S1), jnp.float32(0.0))
    p_ref[...] = jnp.dot(pool, a1, preferred_element_type=jnp.float32)
    m1_ref[...] = jnp.dot(pool, h1, preferred_element_type=jnp.float32)


def _t2_body(h0_ref, m1_ref, p_ref, ws1_ref, wn1_ref, b1_ref,
             ws2_ref, wn2_ref, b2_ref, fcw_ref, fcb_ref, out_ref):
    a0 = h0_ref[...] @ ws1_ref[...] + m1_ref[...] @ wn1_ref[...] + b1_ref[...]
    a0 = jnp.maximum(a0, 0.0)
    e = a0 @ ws2_ref[...] + p_ref[...] @ wn2_ref[...] + b2_ref[...]
    e = jnp.maximum(e, 0.0)
    logits = jnp.maximum(e @ fcw_ref[...] + fcb_ref[...], 0.0)
    m = jnp.max(logits, axis=1, keepdims=True)
    ex = jnp.exp(logits - m)
    out_ref[...] = ex / jnp.sum(ex, axis=1, keepdims=True)


def kernel(batchNodes, features, adj, W_self1, W_neigh1, b1,
           W_self2, W_neigh2, b2, fcW, fcb):
    f32 = jnp.float32
    adj_packed = adj.reshape(N_NODES * MAX_DEG // 128, 128)
    fu = jax.lax.bitcast_convert_type(features, jnp.uint32)
    rnd = jnp.uint32(0x8000)
    features_bf = jax.lax.bitcast_convert_type(
        ((fu[:, : D // 2] + rnd) >> 16)
        | ((fu[:, D // 2:] + rnd) & jnp.uint32(0xFFFF0000)),
        jnp.int32)
    h0, h1, m2 = _sc_gather(batchNodes, adj_packed, features, features_bf)

    b1_2d = b1.reshape(1, D)
    b2_2d = b2.reshape(1, D)
    fcb_2d = fcb.reshape(1, NCLS)

    m1, p = pl.pallas_call(
        _t1_body,
        grid=(NW,),
        in_specs=[
            pl.BlockSpec((L1T, D // 2), lambda i: (i, 0)),
            pl.BlockSpec((L1T, D), lambda i: (i, 0)),
            pl.BlockSpec((D, D), lambda i: (0, 0)),
            pl.BlockSpec((D, D), lambda i: (0, 0)),
            pl.BlockSpec((1, D), lambda i: (0, 0)),
        ],
        out_specs=[
            pl.BlockSpec((BT, D), lambda i: (i, 0)),
            pl.BlockSpec((BT, D), lambda i: (i, 0)),
        ],
        out_shape=[
            jax.ShapeDtypeStruct((B, D), f32),
            jax.ShapeDtypeStruct((B, D), f32),
        ],
    )(h1, m2, W_self1, W_neigh1, b1_2d)

    out = pl.pallas_call(
        _t2_body,
        out_shape=jax.ShapeDtypeStruct((B, NCLS), f32),
    )(h0, m1, p, W_self1, W_neigh1, b1_2d, W_self2, W_neigh2, b2_2d,
      fcW, fcb_2d)
    return out


# speedup vs baseline: 2.9336x; 1.1188x over previous
import jax
import jax.numpy as jnp
from jax import lax
from jax.experimental import pallas as pl
from jax.experimental.pallas import tpu as pltpu
from jax.experimental.pallas import tpu_sc as plsc

N_NODES = 50000
D = 256
B = 1024
S1 = 25
S2 = 10
MAX_DEG = 32
NCLS = 64

NW = 32
BT = B // NW
L1T = BT * S1
CH = 16
NCH = L1T // CH
L2CH = CH * S2
LANES = 16
ADJ_FOLD = 128 // MAX_DEG


def _sc_body(bn_hbm, adj_hbm, feat_hbm, featb_hbm, h0_hbm, h1_hbm, m2_hbm,
             ids_b, rowb, adj_b, h0_buf, l1ids, m2_buf0, m2_buf1,
             rowc0, rowc1, colc0, colc1, adj_c0, adj_c1,
             h1_buf0, h1_buf1, l2c0, l2c1, f2_buf0, f2_buf1,
             sem_a0, sem_a1, sem_f0, sem_f1, sem_h0, sem_h1,
             sem_ho0, sem_ho1, sem_mo0, sem_mo1):
    rowc = (rowc0, rowc1)
    colc = (colc0, colc1)
    adj_c = (adj_c0, adj_c1)
    h1_buf = (h1_buf0, h1_buf1)
    m2_buf = (m2_buf0, m2_buf1)
    l2c = (l2c0, l2c1)
    f2_buf = (f2_buf0, f2_buf1)
    sem_a = (sem_a0, sem_a1)
    sem_f = (sem_f0, sem_f1)
    sem_h = (sem_h0, sem_h1)
    sem_ho = (sem_ho0, sem_ho1)
    sem_mo = (sem_mo0, sem_mo1)

    wid = lax.axis_index("s") * 2 + lax.axis_index("c")
    base_b = pl.multiple_of(wid * BT, BT)
    iot = lax.iota(jnp.int32, LANES)

    pltpu.sync_copy(bn_hbm.at[pl.ds(base_b, BT)], ids_b)
    for j in range(BT // LANES):
        v = ids_b[pl.ds(j * LANES, LANES)]
        rowb[pl.ds(j * LANES, LANES)] = v // ADJ_FOLD
    cp_adj_b = pltpu.async_copy(adj_hbm.at[rowb], adj_b, sem_a0)
    cp_h0 = pltpu.async_copy(feat_hbm.at[ids_b], h0_buf, sem_h0)
    cp_adj_b.wait()
    for j in range(L1T // LANES):
        p = j * LANES + iot
        s = p // S1
        node = plsc.load_gather(ids_b, [s])
        col = (node % ADJ_FOLD) * MAX_DEG + p % S1
        l1ids[pl.ds(j * LANES, LANES)] = plsc.load_gather(adj_b, [s, col])
    cp_h0.wait()
    pltpu.sync_copy(h0_buf, h0_hbm.at[pl.ds(base_b, BT)])

    def start_adj(c, b):
        cs = jnp.minimum(c, NCH - 1)
        v = l1ids[pl.ds(cs * CH, CH)]
        rowc[b][...] = v // ADJ_FOLD
        colc[b][...] = (v % ADJ_FOLD) * MAX_DEG
        pltpu.async_copy(adj_hbm.at[rowc[b]], adj_c[b], sem_a[b])

    def wait_adj(b):
        pltpu.make_async_copy(adj_hbm.at[rowc[b]], adj_c[b], sem_a[b]).wait()

    def start_f2(c, b, wait_ho=True):
        cs = jnp.minimum(c, NCH - 1)
        for j in range(L2CH // LANES):
            p = j * LANES + iot
            s = p // S2
            col = plsc.load_gather(colc[b], [s]) + p % S2
            l2c[b][pl.ds(j * LANES, LANES)] = plsc.load_gather(adj_c[b], [s, col])
        pltpu.async_copy(featb_hbm.at[l2c[b]], f2_buf[b], sem_f[b])
        if wait_ho:
            pltpu.make_async_copy(h1_hbm.at[pl.ds(0, CH)], h1_buf[b],
                                  sem_ho[b]).wait()
        pltpu.async_copy(featb_hbm.at[l1ids.at[pl.ds(cs * CH, CH)]],
                         h1_buf[b], sem_h[b])

    def finish(c, b, wait_mo=True):
        row0 = pl.multiple_of(wid * L1T + c * CH, CH)
        pltpu.make_async_copy(featb_hbm.at[l1ids.at[pl.ds(c * CH, CH)]],
                              h1_buf[b], sem_h[b]).wait()
        pltpu.async_copy(h1_buf[b], h1_hbm.at[pl.ds(row0, CH)], sem_ho[b])
        if wait_mo:
            pltpu.make_async_copy(m2_hbm.at[pl.ds(0, CH)], m2_buf[b],
                                  sem_mo[b]).wait()
        pltpu.make_async_copy(featb_hbm.at[l2c[b]], f2_buf[b], sem_f[b]).wait()

        hi_mask = jnp.int32(-65536)
        inv = jnp.float32(1.0 / S2)

        def seg(s2, carry2):
            for j in range(D // 32):
                r0 = s2 * S2
                vi = f2_buf[b][r0, pl.ds(j * LANES, LANES)]
                acc_lo = plsc.bitcast(vi << 16, jnp.float32)
                acc_hi = plsc.bitcast(vi & hi_mask, jnp.float32)
                for t in range(1, S2):
                    vi = f2_buf[b][r0 + t, pl.ds(j * LANES, LANES)]
                    acc_lo = acc_lo + plsc.bitcast(vi << 16, jnp.float32)
                    acc_hi = acc_hi + plsc.bitcast(vi & hi_mask, jnp.float32)
                m2_buf[b][s2, pl.ds(j * LANES, LANES)] = acc_lo * inv
                m2_buf[b][s2, pl.ds(D // 2 + j * LANES, LANES)] = acc_hi * inv
            return carry2

        lax.fori_loop(0, CH, seg, 0)
        pltpu.async_copy(m2_buf[b], m2_hbm.at[pl.ds(row0, CH)], sem_mo[b])

    start_adj(0, 0)
    wait_adj(0)
    start_f2(0, 0, wait_ho=False)
    start_adj(1, 1)

    def pair_body(c0, wait_first):
        wait_adj(1)
        start_f2(c0 + 1, 1, wait_ho=wait_first)
        start_adj(c0 + 2, 0)
        finish(c0, 0, wait_mo=wait_first)
        wait_adj(0)
        start_f2(c0 + 2, 0)
        start_adj(c0 + 3, 1)
        finish(c0 + 1, 1, wait_mo=wait_first)

    pair_body(jnp.int32(0), False)

    def pair(g, carry):
        pair_body(g * 2, True)
        return carry

    lax.fori_loop(1, NCH // 2, pair, 0)
    pltpu.make_async_copy(featb_hbm.at[l1ids.at[pl.ds((NCH - 1) * CH, CH)]],
                          h1_buf[0], sem_h[0]).wait()
    pltpu.make_async_copy(featb_hbm.at[l2c[0]], f2_buf[0], sem_f[0]).wait()
    wait_adj(1)
    pltpu.make_async_copy(h1_hbm.at[pl.ds(0, CH)], h1_buf[1], sem_ho[1]).wait()
    pltpu.make_async_copy(m2_hbm.at[pl.ds(0, CH)], m2_buf[0], sem_mo[0]).wait()
    pltpu.make_async_copy(m2_hbm.at[pl.ds(0, CH)], m2_buf[1], sem_mo[1]).wait()


def _sc_gather(batch_nodes, adj_packed, features, features_bf):
    mesh = plsc.VectorSubcoreMesh(core_axis_name="c", subcore_axis_name="s")
    f32, i32 = jnp.float32, jnp.int32
    fn = pl.kernel(
        _sc_body,
        mesh=mesh,
        compiler_params=pltpu.CompilerParams(needs_layout_passes=False),
        out_type=[
            jax.ShapeDtypeStruct((B, D), f32),
            jax.ShapeDtypeStruct((B * S1, D // 2), i32),
            jax.ShapeDtypeStruct((B * S1, D), f32),
        ],
        scratch_types=[
            pltpu.VMEM((BT,), i32),
            pltpu.VMEM((BT,), i32),
            pltpu.VMEM((BT, 128), i32),
            pltpu.VMEM((BT, D), f32),
            pltpu.VMEM((L1T,), i32),
            pltpu.VMEM((CH, D), f32),
            pltpu.VMEM((CH, D), f32),
            pltpu.VMEM((CH,), i32),
            pltpu.VMEM((CH,), i32),
            pltpu.VMEM((CH,), i32),
            pltpu.VMEM((CH,), i32),
            pltpu.VMEM((CH, 128), i32),
            pltpu.VMEM((CH, 128), i32),
            pltpu.VMEM((CH, D // 2), i32),
            pltpu.VMEM((CH, D // 2), i32),
            pltpu.VMEM((L2CH,), i32),
            pltpu.VMEM((L2CH,), i32),
            pltpu.VMEM((L2CH, D // 2), i32),
            pltpu.VMEM((L2CH, D // 2), i32),
            pltpu.SemaphoreType.DMA,
            pltpu.SemaphoreType.DMA,
            pltpu.SemaphoreType.DMA,
            pltpu.SemaphoreType.DMA,
            pltpu.SemaphoreType.DMA,
            pltpu.SemaphoreType.DMA,
            pltpu.SemaphoreType.DMA,
            pltpu.SemaphoreType.DMA,
            pltpu.SemaphoreType.DMA,
            pltpu.SemaphoreType.DMA,
        ],
    )
    return fn(batch_nodes, adj_packed, features, features_bf)


def _pack_body(f_ref, out_ref):
    fu = jax.lax.bitcast_convert_type(f_ref[...], jnp.uint32)
    rnd = jnp.uint32(0x8000)
    out_ref[...] = jax.lax.bitcast_convert_type(
        ((fu[:, : D // 2] + rnd) >> 16)
        | ((fu[:, D // 2:] + rnd) & jnp.uint32(0xFFFF0000)),
        jnp.int32)


def _t1_body(h1p_ref, m2_ref, ws_ref, wn_ref, b_ref, m1_ref, p_ref):
    vi = h1p_ref[...]
    h1 = jnp.concatenate(
        [
            jax.lax.bitcast_convert_type(vi << 16, jnp.float32),
            jax.lax.bitcast_convert_type(vi & jnp.int32(-65536), jnp.float32),
        ],
        axis=1,
    )
    a1 = h1 @ ws_ref[...] + m2_ref[...] @ wn_ref[...] + b_ref[...]
    a1 = jnp.maximum(a1, 0.0)
    rows = lax.broadcasted_iota(jnp.int32, (BT, L1T), 0)
    segs = lax.broadcasted_iota(jnp.int32, (BT, L1T), 1) // S1
    pool = jnp.where(rows == segs, jnp.float32(1.0 / S1), jnp.float32(0.0))
    p_ref[...] = jnp.dot(pool, a1, preferred_element_type=jnp.float32)
    m1_ref[...] = jnp.dot(pool, h1, preferred_element_type=jnp.float32)


def _t2_body(h0_ref, m1_ref, p_ref, ws1_ref, wn1_ref, b1_ref,
             ws2_ref, wn2_ref, b2_ref, fcw_ref, fcb_ref, out_ref):
    a0 = h0_ref[...] @ ws1_ref[...] + m1_ref[...] @ wn1_ref[...] + b1_ref[...]
    a0 = jnp.maximum(a0, 0.0)
    e = a0 @ ws2_ref[...] + p_ref[...] @ wn2_ref[...] + b2_ref[...]
    e = jnp.maximum(e, 0.0)
    logits = jnp.maximum(e @ fcw_ref[...] + fcb_ref[...], 0.0)
    m = jnp.max(logits, axis=1, keepdims=True)
    ex = jnp.exp(logits - m)
    out_ref[...] = ex / jnp.sum(ex, axis=1, keepdims=True)


def kernel(batchNodes, features, adj, W_self1, W_neigh1, b1,
           W_self2, W_neigh2, b2, fcW, fcb):
    f32 = jnp.float32
    adj_packed = adj.reshape(N_NODES * MAX_DEG // 128, 128)
    features_bf = pl.pallas_call(
        _pack_body,
        grid=(25,),
        in_specs=[pl.BlockSpec((N_NODES // 25, D), lambda i: (i, 0))],
        out_specs=pl.BlockSpec((N_NODES // 25, D // 2), lambda i: (i, 0)),
        out_shape=jax.ShapeDtypeStruct((N_NODES, D // 2), jnp.int32),
    )(features)
    h0, h1, m2 = _sc_gather(batchNodes, adj_packed, features, features_bf)

    b1_2d = b1.reshape(1, D)
    b2_2d = b2.reshape(1, D)
    fcb_2d = fcb.reshape(1, NCLS)

    m1, p = pl.pallas_call(
        _t1_body,
        grid=(NW,),
        in_specs=[
            pl.BlockSpec((L1T, D // 2), lambda i: (i, 0)),
            pl.BlockSpec((L1T, D), lambda i: (i, 0)),
            pl.BlockSpec((D, D), lambda i: (0, 0)),
            pl.BlockSpec((D, D), lambda i: (0, 0)),
            pl.BlockSpec((1, D), lambda i: (0, 0)),
        ],
        out_specs=[
            pl.BlockSpec((BT, D), lambda i: (i, 0)),
            pl.BlockSpec((BT, D), lambda i: (i, 0)),
        ],
        out_shape=[
            jax.ShapeDtypeStruct((B, D), f32),
            jax.ShapeDtypeStruct((B, D), f32),
        ],
    )(h1, m2, W_self1, W_neigh1, b1_2d)

    out = pl.pallas_call(
        _t2_body,
        out_shape=jax.ShapeDtypeStruct((B, NCLS), f32),
    )(h0, m1, p, W_self1, W_neigh1, b1_2d, W_self2, W_neigh2, b2_2d,
      fcW, fcb_2d)
    return out


# speedup vs baseline: 2.9391x; 1.0019x over previous
import jax
import jax.numpy as jnp
from jax import lax
from jax.experimental import pallas as pl
from jax.experimental.pallas import tpu as pltpu
from jax.experimental.pallas import tpu_sc as plsc

N_NODES = 50000
D = 256
B = 1024
S1 = 25
S2 = 10
MAX_DEG = 32
NCLS = 64

NW = 32
BT = B // NW
L1T = BT * S1
CH = 16
NCH = L1T // CH
L2CH = CH * S2
LANES = 16
ADJ_FOLD = 128 // MAX_DEG


def _sc_body(bn_hbm, adj_hbm, feat_hbm, featb_hbm, h0_hbm, h1_hbm, m2_hbm,
             ids_b, rowb, adj_b, h0_buf, l1ids, m2_buf0, m2_buf1,
             rowc0, rowc1, colc0, colc1, adj_c0, adj_c1,
             h1_buf0, h1_buf1, l2c0, l2c1, f2_buf0, f2_buf1,
             sem_a0, sem_a1, sem_f0, sem_f1, sem_h0, sem_h1,
             sem_ho0, sem_ho1, sem_mo0, sem_mo1):
    rowc = (rowc0, rowc1)
    colc = (colc0, colc1)
    adj_c = (adj_c0, adj_c1)
    h1_buf = (h1_buf0, h1_buf1)
    m2_buf = (m2_buf0, m2_buf1)
    l2c = (l2c0, l2c1)
    f2_buf = (f2_buf0, f2_buf1)
    sem_a = (sem_a0, sem_a1)
    sem_f = (sem_f0, sem_f1)
    sem_h = (sem_h0, sem_h1)
    sem_ho = (sem_ho0, sem_ho1)
    sem_mo = (sem_mo0, sem_mo1)

    wid = lax.axis_index("s") * 2 + lax.axis_index("c")
    base_b = pl.multiple_of(wid * BT, BT)
    iot = lax.iota(jnp.int32, LANES)

    pltpu.sync_copy(bn_hbm.at[pl.ds(base_b, BT)], ids_b)
    for j in range(BT // LANES):
        v = ids_b[pl.ds(j * LANES, LANES)]
        rowb[pl.ds(j * LANES, LANES)] = v // ADJ_FOLD
    cp_adj_b = pltpu.async_copy(adj_hbm.at[rowb], adj_b, sem_a0)
    cp_h0 = pltpu.async_copy(feat_hbm.at[ids_b], h0_buf, sem_h0)
    cp_adj_b.wait()
    for j in range(L1T // LANES):
        p = j * LANES + iot
        s = p // S1
        node = plsc.load_gather(ids_b, [s])
        col = (node % ADJ_FOLD) * MAX_DEG + p % S1
        l1ids[pl.ds(j * LANES, LANES)] = plsc.load_gather(adj_b, [s, col])
    cp_h0.wait()
    pltpu.sync_copy(h0_buf, h0_hbm.at[pl.ds(base_b, BT)])

    def start_adj(c, b):
        cs = jnp.minimum(c, NCH - 1)
        v = l1ids[pl.ds(cs * CH, CH)]
        rowc[b][...] = v // ADJ_FOLD
        colc[b][...] = (v % ADJ_FOLD) * MAX_DEG
        pltpu.async_copy(adj_hbm.at[rowc[b]], adj_c[b], sem_a[b])

    def wait_adj(b):
        pltpu.make_async_copy(adj_hbm.at[rowc[b]], adj_c[b], sem_a[b]).wait()

    def start_f2(c, b, wait_ho=True):
        cs = jnp.minimum(c, NCH - 1)
        for j in range(L2CH // LANES):
            p = j * LANES + iot
            s = p // S2
            col = plsc.load_gather(colc[b], [s]) + p % S2
            l2c[b][pl.ds(j * LANES, LANES)] = plsc.load_gather(adj_c[b], [s, col])
        pltpu.async_copy(featb_hbm.at[l2c[b]], f2_buf[b], sem_f[b])
        if wait_ho:
            pltpu.make_async_copy(h1_hbm.at[pl.ds(0, CH)], h1_buf[b],
                                  sem_ho[b]).wait()
        pltpu.async_copy(featb_hbm.at[l1ids.at[pl.ds(cs * CH, CH)]],
                         h1_buf[b], sem_h[b])

    def finish(c, b, wait_mo=True):
        row0 = pl.multiple_of(wid * L1T + c * CH, CH)
        pltpu.make_async_copy(featb_hbm.at[l1ids.at[pl.ds(c * CH, CH)]],
                              h1_buf[b], sem_h[b]).wait()
        pltpu.async_copy(h1_buf[b], h1_hbm.at[pl.ds(row0, CH)], sem_ho[b])
        if wait_mo:
            pltpu.make_async_copy(m2_hbm.at[pl.ds(0, CH)], m2_buf[b],
                                  sem_mo[b]).wait()
        pltpu.make_async_copy(featb_hbm.at[l2c[b]], f2_buf[b], sem_f[b]).wait()

        hi_mask = jnp.int32(-65536)
        inv = jnp.float32(1.0 / S2)

        def seg(s2, carry2):
            for j in range(D // 32):
                r0 = s2 * S2
                vi = f2_buf[b][r0, pl.ds(j * LANES, LANES)]
                acc_lo = plsc.bitcast(vi << 16, jnp.float32)
                acc_hi = plsc.bitcast(vi & hi_mask, jnp.float32)
                for t in range(1, S2):
                    vi = f2_buf[b][r0 + t, pl.ds(j * LANES, LANES)]
                    acc_lo = acc_lo + plsc.bitcast(vi << 16, jnp.float32)
                    acc_hi = acc_hi + plsc.bitcast(vi & hi_mask, jnp.float32)
                m2_buf[b][s2, pl.ds(j * LANES, LANES)] = acc_lo * inv
                m2_buf[b][s2, pl.ds(D // 2 + j * LANES, LANES)] = acc_hi * inv
            return carry2

        lax.fori_loop(0, CH, seg, 0)
        pltpu.async_copy(m2_buf[b], m2_hbm.at[pl.ds(row0, CH)], sem_mo[b])

    start_adj(0, 0)
    wait_adj(0)
    start_f2(0, 0, wait_ho=False)
    start_adj(1, 1)

    def pair_body(c0, wait_first):
        wait_adj(1)
        start_f2(c0 + 1, 1, wait_ho=wait_first)
        start_adj(c0 + 2, 0)
        finish(c0, 0, wait_mo=wait_first)
        wait_adj(0)
        start_f2(c0 + 2, 0)
        start_adj(c0 + 3, 1)
        finish(c0 + 1, 1, wait_mo=wait_first)

    pair_body(jnp.int32(0), False)

    def pair(g, carry):
        pair_body(g * 2, True)
        return carry

    lax.fori_loop(1, NCH // 2, pair, 0)
    pltpu.make_async_copy(featb_hbm.at[l1ids.at[pl.ds((NCH - 1) * CH, CH)]],
                          h1_buf[0], sem_h[0]).wait()
    pltpu.make_async_copy(featb_hbm.at[l2c[0]], f2_buf[0], sem_f[0]).wait()
    wait_adj(1)
    pltpu.make_async_copy(h1_hbm.at[pl.ds(0, CH)], h1_buf[1], sem_ho[1]).wait()
    pltpu.make_async_copy(m2_hbm.at[pl.ds(0, CH)], m2_buf[0], sem_mo[0]).wait()
    pltpu.make_async_copy(m2_hbm.at[pl.ds(0, CH)], m2_buf[1], sem_mo[1]).wait()


def _sc_gather(batch_nodes, adj_packed, features, features_bf):
    mesh = plsc.VectorSubcoreMesh(core_axis_name="c", subcore_axis_name="s")
    f32, i32 = jnp.float32, jnp.int32
    fn = pl.kernel(
        _sc_body,
        mesh=mesh,
        compiler_params=pltpu.CompilerParams(needs_layout_passes=False),
        out_type=[
            jax.ShapeDtypeStruct((B, D), f32),
            jax.ShapeDtypeStruct((B * S1, D // 2), i32),
            jax.ShapeDtypeStruct((B * S1, D), f32),
        ],
        scratch_types=[
            pltpu.VMEM((BT,), i32),
            pltpu.VMEM((BT,), i32),
            pltpu.VMEM((BT, 128), i32),
            pltpu.VMEM((BT, D), f32),
            pltpu.VMEM((L1T,), i32),
            pltpu.VMEM((CH, D), f32),
            pltpu.VMEM((CH, D), f32),
            pltpu.VMEM((CH,), i32),
            pltpu.VMEM((CH,), i32),
            pltpu.VMEM((CH,), i32),
            pltpu.VMEM((CH,), i32),
            pltpu.VMEM((CH, 128), i32),
            pltpu.VMEM((CH, 128), i32),
            pltpu.VMEM((CH, D // 2), i32),
            pltpu.VMEM((CH, D // 2), i32),
            pltpu.VMEM((L2CH,), i32),
            pltpu.VMEM((L2CH,), i32),
            pltpu.VMEM((L2CH, D // 2), i32),
            pltpu.VMEM((L2CH, D // 2), i32),
            pltpu.SemaphoreType.DMA,
            pltpu.SemaphoreType.DMA,
            pltpu.SemaphoreType.DMA,
            pltpu.SemaphoreType.DMA,
            pltpu.SemaphoreType.DMA,
            pltpu.SemaphoreType.DMA,
            pltpu.SemaphoreType.DMA,
            pltpu.SemaphoreType.DMA,
            pltpu.SemaphoreType.DMA,
            pltpu.SemaphoreType.DMA,
        ],
    )
    return fn(batch_nodes, adj_packed, features, features_bf)


def _pack_body(f_ref, out_ref):
    fu = jax.lax.bitcast_convert_type(f_ref[...], jnp.uint32)
    rnd = jnp.uint32(0x8000)
    out_ref[...] = jax.lax.bitcast_convert_type(
        ((fu[:, : D // 2] + rnd) >> 16)
        | ((fu[:, D // 2:] + rnd) & jnp.uint32(0xFFFF0000)),
        jnp.int32)


def _t1_body(h1p_ref, m2_ref, ws_ref, wn_ref, b_ref, m1_ref, p_ref):
    vi = h1p_ref[...]
    h1 = jnp.concatenate(
        [
            jax.lax.bitcast_convert_type(vi << 16, jnp.float32),
            jax.lax.bitcast_convert_type(vi & jnp.int32(-65536), jnp.float32),
        ],
        axis=1,
    )
    bf = jnp.bfloat16
    a1 = (jnp.dot(h1.astype(bf), ws_ref[...].astype(bf),
                  preferred_element_type=jnp.float32)
          + jnp.dot(m2_ref[...].astype(bf), wn_ref[...].astype(bf),
                    preferred_element_type=jnp.float32)
          + b_ref[...])
    a1 = jnp.maximum(a1, 0.0)
    rows = lax.broadcasted_iota(jnp.int32, (BT, L1T), 0)
    segs = lax.broadcasted_iota(jnp.int32, (BT, L1T), 1) // S1
    pool = jnp.where(rows == segs, jnp.float32(1.0 / S1), jnp.float32(0.0))
    p_ref[...] = jnp.dot(pool.astype(bf), a1.astype(bf),
                         preferred_element_type=jnp.float32)
    m1_ref[...] = jnp.dot(pool.astype(bf), h1.astype(bf),
                          preferred_element_type=jnp.float32)


def _t2_body(h0_ref, m1_ref, p_ref, ws1_ref, wn1_ref, b1_ref,
             ws2_ref, wn2_ref, b2_ref, fcw_ref, fcb_ref, out_ref):
    a0 = h0_ref[...] @ ws1_ref[...] + m1_ref[...] @ wn1_ref[...] + b1_ref[...]
    a0 = jnp.maximum(a0, 0.0)
    e = a0 @ ws2_ref[...] + p_ref[...] @ wn2_ref[...] + b2_ref[...]
    e = jnp.maximum(e, 0.0)
    logits = jnp.maximum(e @ fcw_ref[...] + fcb_ref[...], 0.0)
    m = jnp.max(logits, axis=1, keepdims=True)
    ex = jnp.exp(logits - m)
    out_ref[...] = ex / jnp.sum(ex, axis=1, keepdims=True)


def kernel(batchNodes, features, adj, W_self1, W_neigh1, b1,
           W_self2, W_neigh2, b2, fcW, fcb):
    f32 = jnp.float32
    adj_packed = adj.reshape(N_NODES * MAX_DEG // 128, 128)
    features_bf = pl.pallas_call(
        _pack_body,
        grid=(25,),
        in_specs=[pl.BlockSpec((N_NODES // 25, D), lambda i: (i, 0))],
        out_specs=pl.BlockSpec((N_NODES // 25, D // 2), lambda i: (i, 0)),
        out_shape=jax.ShapeDtypeStruct((N_NODES, D // 2), jnp.int32),
    )(features)
    h0, h1, m2 = _sc_gather(batchNodes, adj_packed, features, features_bf)

    b1_2d = b1.reshape(1, D)
    b2_2d = b2.reshape(1, D)
    fcb_2d = fcb.reshape(1, NCLS)

    m1, p = pl.pallas_call(
        _t1_body,
        grid=(NW,),
        in_specs=[
            pl.BlockSpec((L1T, D // 2), lambda i: (i, 0)),
            pl.BlockSpec((L1T, D), lambda i: (i, 0)),
            pl.BlockSpec((D, D), lambda i: (0, 0)),
            pl.BlockSpec((D, D), lambda i: (0, 0)),
            pl.BlockSpec((1, D), lambda i: (0, 0)),
        ],
        out_specs=[
            pl.BlockSpec((BT, D), lambda i: (i, 0)),
            pl.BlockSpec((BT, D), lambda i: (i, 0)),
        ],
        out_shape=[
            jax.ShapeDtypeStruct((B, D), f32),
            jax.ShapeDtypeStruct((B, D), f32),
        ],
    )(h1, m2, W_self1, W_neigh1, b1_2d)

    out = pl.pallas_call(
        _t2_body,
        out_shape=jax.ShapeDtypeStruct((B, NCLS), f32),
    )(h0, m1, p, W_self1, W_neigh1, b1_2d, W_self2, W_neigh2, b2_2d,
      fcW, fcb_2d)
    return out
